# baseline (speedup 1.0000x reference)
import functools

import jax
import jax.numpy as jnp
from jax import lax
from jax.experimental import pallas as pl
from jax.experimental.pallas import tpu as pltpu
from jax.experimental.pallas import tpu_sc as plsc

_MAX_POS = 1024
_LANES = 16
_CH = 128
_NBUF = 4


@functools.lru_cache(maxsize=None)
def _build(n, vocab, d):
    info = plsc.get_sparse_core_info()
    nw = info.num_cores * info.num_subcores
    nch = -(-n // (nw * _CH))
    bpw = nch * _CH
    assert n % 8 == 0 and bpw % 8 == 0 and n >= bpw
    assert nch % _NBUF == 1 and nch >= 2 * _NBUF

    mesh = plsc.VectorSubcoreMesh(core_axis_name="c", subcore_axis_name="s")

    def body(table_hbm, idx_hbm, out_hbm, idx_v, buf_v, table_sh, gsem, wsem, ssem):
        sid = lax.axis_index("s")
        wid = sid * info.num_cores + lax.axis_index("c")
        base = jnp.minimum(wid * bpw, n - bpw)

        @pl.when(sid == 0)
        def _():
            pltpu.async_copy(table_hbm, table_sh, ssem)

        pltpu.sync_copy(idx_hbm.at[pl.ds(pl.multiple_of(base, 8), bpw)], idx_v)

        def clamp(i, carry):
            s = pl.ds(i * _LANES, _LANES)
            idx_v[s] = jnp.minimum(jnp.maximum(idx_v[s], 0), _MAX_POS)
            return carry

        lax.fori_loop(0, bpw // _LANES, clamp, 0)

        @pl.when(sid == 0)
        def _():
            pltpu.make_async_copy(table_hbm, table_sh, ssem).wait()

        plsc.subcore_barrier()

        def fire_gather(ck, b):
            return pltpu.async_copy(
                table_sh.at[idx_v.at[pl.ds(ck * _CH, _CH)]],
                buf_v.at[b],
                gsem.at[b],
            )

        def fire_write(ck, b):
            off = pl.multiple_of(base + ck * _CH, 8)
            return pltpu.async_copy(
                buf_v.at[b],
                out_hbm.at[pl.ds(off, _CH)],
                wsem.at[b],
            )

        def wait_gather(b):
            pltpu.make_async_copy(
                table_sh.at[idx_v.at[pl.ds(0, _CH)]], buf_v.at[b], gsem.at[b]
            ).wait()

        def wait_write(b):
            pltpu.make_async_copy(
                buf_v.at[b], out_hbm.at[pl.ds(0, _CH)], wsem.at[b]
            ).wait()

        fire_gather(0, 0)
        for ck in range(_NBUF):
            if ck == _NBUF - 1:
                wait_write(0)
            fire_gather(ck + 1, (ck + 1) % _NBUF)
            wait_gather(ck % _NBUF)
            fire_write(ck, ck % _NBUF)

        def group(g, carry):
            for b in range(_NBUF):
                ck = g * _NBUF + b
                wait_write((b + 1) % _NBUF)
                fire_gather(ck + 1, (b + 1) % _NBUF)
                wait_gather(b)
                fire_write(ck, b)
            return carry

        lax.fori_loop(1, (nch - 1) // _NBUF, group, 0)

        wait_gather(0)
        fire_write(nch - 1, 0)
        for b in range(_NBUF):
            wait_write(b)

    return pl.kernel(
        body,
        mesh=mesh,
        out_type=jax.ShapeDtypeStruct((n, d), jnp.float32),
        scratch_types=[
            pltpu.VMEM((bpw,), jnp.int32),
            pltpu.VMEM((_NBUF, _CH, d), jnp.float32),
            pltpu.VMEM_SHARED((vocab, d), jnp.float32),
            pltpu.SemaphoreType.DMA((_NBUF,)),
            pltpu.SemaphoreType.DMA((_NBUF,)),
            pltpu.SemaphoreType.DMA,
        ],
    )


def kernel(node_idx, pos_embedding_weight):
    n = node_idx.shape[0]
    vocab, d = pos_embedding_weight.shape
    f = _build(n, vocab, d)
    return f(pos_embedding_weight, node_idx.astype(jnp.int32))


# speedup vs baseline: 1.0252x; 1.0252x over previous
import functools

import jax
import jax.numpy as jnp
from jax import lax
from jax.experimental import pallas as pl
from jax.experimental.pallas import tpu as pltpu
from jax.experimental.pallas import tpu_sc as plsc

_MAX_POS = 1024
_LANES = 16
_CH = 112
_NBUF = 4
_LAG = 2


@functools.lru_cache(maxsize=None)
def _build(n, vocab, d):
    info = plsc.get_sparse_core_info()
    nw = info.num_cores * info.num_subcores
    nch = -(-n // (nw * _CH))
    bpw = nch * _CH
    assert n % 8 == 0 and bpw % 8 == 0 and bpw % _LANES == 0 and n >= bpw

    mesh = plsc.VectorSubcoreMesh(core_axis_name="c", subcore_axis_name="s")

    def body(table_hbm, idx_hbm, out_hbm, idx_v, buf_v, table_sh, gsem, wsem, ssem):
        sid = lax.axis_index("s")
        wid = sid * info.num_cores + lax.axis_index("c")
        base = jnp.minimum(wid * bpw, n - bpw)

        @pl.when(sid == 0)
        def _():
            pltpu.async_copy(table_hbm, table_sh, ssem)

        pltpu.sync_copy(idx_hbm.at[pl.ds(pl.multiple_of(base, 8), bpw)], idx_v)

        def clamp(i, carry):
            s = pl.ds(i * _LANES, _LANES)
            idx_v[s] = jnp.minimum(jnp.maximum(idx_v[s], 0), _MAX_POS)
            return carry

        lax.fori_loop(0, bpw // _LANES, clamp, 0)

        @pl.when(sid == 0)
        def _():
            pltpu.make_async_copy(table_hbm, table_sh, ssem).wait()

        plsc.subcore_barrier()

        def fire_gather(ck):
            b = ck % _NBUF
            return pltpu.async_copy(
                table_sh.at[idx_v.at[pl.ds(ck * _CH, _CH)]],
                buf_v.at[b],
                gsem.at[b],
            )

        def fire_write(ck):
            b = ck % _NBUF
            off = pl.multiple_of(base + ck * _CH, 8)
            return pltpu.async_copy(
                buf_v.at[b],
                out_hbm.at[pl.ds(off, _CH)],
                wsem.at[b],
            )

        gathers = {}
        writes = {}
        for t in range(nch + _LAG):
            if t < nch:
                if t >= _NBUF:
                    writes.pop(t - _NBUF).wait()
                gathers[t] = fire_gather(t)
            j = t - _LAG
            if j >= 0:
                gathers.pop(j).wait()
                writes[j] = fire_write(j)
        for j in sorted(writes):
            writes.pop(j).wait()

    return pl.kernel(
        body,
        mesh=mesh,
        out_type=jax.ShapeDtypeStruct((n, d), jnp.float32),
        scratch_types=[
            pltpu.VMEM((bpw,), jnp.int32),
            pltpu.VMEM((_NBUF, _CH, d), jnp.float32),
            pltpu.VMEM_SHARED((vocab, d), jnp.float32),
            pltpu.SemaphoreType.DMA((_NBUF,)),
            pltpu.SemaphoreType.DMA((_NBUF,)),
            pltpu.SemaphoreType.DMA,
        ],
    )


def kernel(node_idx, pos_embedding_weight):
    n = node_idx.shape[0]
    vocab, d = pos_embedding_weight.shape
    f = _build(n, vocab, d)
    return f(pos_embedding_weight, node_idx.astype(jnp.int32))
